# baseline (speedup 1.0000x reference)
import dataclasses
import functools

import jax
import jax.numpy as jnp
from jax import lax
from jax.experimental import pallas as pl
from jax.experimental.pallas import tpu as pltpu
from jax.experimental.pallas import tpu_sc as plsc

BATCH = 16384
CH = 4096
NC = 2
NS = 16
NW = NC * NS
ROWS_PER_W = BATCH // NW
CHUNK = 4
NCHUNKS = ROWS_PER_W // CHUNK
LANES = 16


def _shuffle_sc(in_flat, perm32):
    mesh = plsc.VectorSubcoreMesh(core_axis_name="c", subcore_axis_name="s")
    cp = pltpu.CompilerParams()
    if "needs_layout_passes" in pltpu.CompilerParams.__dataclass_fields__:
        cp = dataclasses.replace(cp, needs_layout_passes=False)

    @functools.partial(
        pl.kernel,
        mesh=mesh,
        compiler_params=cp,
        out_type=jax.ShapeDtypeStruct((BATCH * CH,), jnp.float32),
        scratch_types=[
            pltpu.VMEM((CH,), jnp.int32),
            pltpu.VMEM((CHUNK * CH,), jnp.float32),
            pltpu.VMEM((CHUNK * CH,), jnp.float32),
            pltpu.VMEM((CHUNK * CH,), jnp.float32),
            pltpu.VMEM((CHUNK * CH,), jnp.float32),
            pltpu.SemaphoreType.DMA,
            pltpu.SemaphoreType.DMA,
            pltpu.SemaphoreType.DMA,
            pltpu.SemaphoreType.DMA,
        ],
    )
    def k(in_hbm, perm_hbm, out_hbm, perm_v, in_a, in_b, out_a, out_b,
          isem_a, isem_b, osem_a, osem_b):
        wid = lax.axis_index("s") * NC + lax.axis_index("c")
        base = wid * ROWS_PER_W * CH
        nelem = CHUNK * CH
        pltpu.sync_copy(perm_hbm, perm_v)

        bufs = ((in_a, isem_a, out_a, osem_a), (in_b, isem_b, out_b, osem_b))

        def in_copy(c, buf, sem):
            return pltpu.make_async_copy(
                in_hbm.at[pl.ds(base + c * nelem, nelem)], buf, sem)

        def out_copy(c, buf, sem):
            return pltpu.make_async_copy(
                buf, out_hbm.at[pl.ds(base + c * nelem, nelem)], sem)

        in_copy(0, in_a, isem_a).start()
        in_copy(1, in_b, isem_b).start()

        @pl.loop(0, NCHUNKS // 2)
        def _(g):
            for b in range(2):
                in_v, isem, out_v, osem = bufs[b]
                c = g * 2 + b
                in_copy(c, in_v, isem).wait()

                @pl.when(g > 0)
                def _():
                    out_copy(c - 2, out_v, osem).wait()

                @plsc.parallel_loop(0, CH, LANES, unroll=4)
                def _(j):
                    pv = perm_v[pl.ds(j, LANES)]
                    for r in range(CHUNK):
                        g16 = plsc.load_gather(in_v, [pv + jnp.int32(r * CH)])
                        out_v[pl.ds(r * CH + j, LANES)] = g16

                out_copy(c, out_v, osem).start()

                @pl.when(c + 2 < NCHUNKS)
                def _():
                    in_copy(c + 2, in_v, isem).start()

        out_copy(NCHUNKS - 2, out_a, osem_a).wait()
        out_copy(NCHUNKS - 1, out_b, osem_b).wait()

    return k(in_flat, perm32)


def kernel(inputs, perm):
    perm32 = perm.astype(jnp.int32)
    out_flat = _shuffle_sc(inputs.reshape(-1), perm32)
    shuffled = out_flat.reshape(BATCH, CH)
    logdet = jnp.zeros((BATCH,), dtype=inputs.dtype)
    return (shuffled, logdet)


# speedup vs baseline: 3.0917x; 3.0917x over previous
import dataclasses
import functools

import jax
import jax.numpy as jnp
from jax import lax
from jax.experimental import pallas as pl
from jax.experimental.pallas import tpu as pltpu
from jax.experimental.pallas import tpu_sc as plsc

BATCH = 16384
CH = 4096
NC = 2
NS = 16
NW = NC * NS
ROWS_PER_W = BATCH // NW
CHUNK = 4
NCHUNKS = ROWS_PER_W // CHUNK
LANES = 16


def _shuffle_sc(inputs, perm32):
    mesh = plsc.VectorSubcoreMesh(core_axis_name="c", subcore_axis_name="s")
    cp = pltpu.CompilerParams()
    if "needs_layout_passes" in pltpu.CompilerParams.__dataclass_fields__:
        cp = dataclasses.replace(cp, needs_layout_passes=False)

    @functools.partial(
        pl.kernel,
        mesh=mesh,
        compiler_params=cp,
        out_type=jax.ShapeDtypeStruct((BATCH, CH), jnp.float32),
        scratch_types=[
            pltpu.VMEM((CH,), jnp.int32),
            pltpu.VMEM((CHUNK, CH), jnp.float32),
            pltpu.VMEM((CHUNK, CH), jnp.float32),
            pltpu.VMEM((CHUNK, CH), jnp.float32),
            pltpu.VMEM((CHUNK, CH), jnp.float32),
            pltpu.SemaphoreType.DMA,
            pltpu.SemaphoreType.DMA,
            pltpu.SemaphoreType.DMA,
            pltpu.SemaphoreType.DMA,
        ],
    )
    def k(in_hbm, perm_hbm, out_hbm, perm_v, in_a, in_b, out_a, out_b,
          isem_a, isem_b, osem_a, osem_b):
        wid = lax.axis_index("s") * NC + lax.axis_index("c")
        base_row = wid * ROWS_PER_W
        pltpu.sync_copy(perm_hbm, perm_v)

        bufs = ((in_a, isem_a, out_a, osem_a), (in_b, isem_b, out_b, osem_b))

        def in_copy(c, buf, sem):
            return pltpu.make_async_copy(
                in_hbm.at[pl.ds(base_row + c * CHUNK, CHUNK)], buf, sem)

        def out_copy(c, buf, sem):
            return pltpu.make_async_copy(
                buf, out_hbm.at[pl.ds(base_row + c * CHUNK, CHUNK)], sem)

        in_copy(0, in_a, isem_a).start()
        in_copy(1, in_b, isem_b).start()

        @pl.loop(0, NCHUNKS // 2)
        def _(g):
            for b in range(2):
                in_v, isem, out_v, osem = bufs[b]
                c = g * 2 + b
                in_copy(c, in_v, isem).wait()

                @pl.when(g > 0)
                def _():
                    out_copy(c - 2, out_v, osem).wait()

                @plsc.parallel_loop(0, CH, LANES, unroll=2)
                def _(j):
                    pv = perm_v[pl.ds(j, LANES)]
                    for r in range(CHUNK):
                        rsplat = jnp.full((LANES,), r, jnp.int32)
                        g16 = plsc.load_gather(in_v, [rsplat, pv])
                        out_v[r, pl.ds(j, LANES)] = g16

                out_copy(c, out_v, osem).start()

                @pl.when(c + 2 < NCHUNKS)
                def _():
                    in_copy(c + 2, in_v, isem).start()

        out_copy(NCHUNKS - 2, out_a, osem_a).wait()
        out_copy(NCHUNKS - 1, out_b, osem_b).wait()

    return k(inputs, perm32)


def kernel(inputs, perm):
    perm32 = perm.astype(jnp.int32)
    shuffled = _shuffle_sc(inputs, perm32)
    logdet = jnp.zeros((BATCH,), dtype=inputs.dtype)
    return (shuffled, logdet)


# speedup vs baseline: 3.0951x; 1.0011x over previous
import dataclasses
import functools

import jax
import jax.numpy as jnp
from jax import lax
from jax.experimental import pallas as pl
from jax.experimental.pallas import tpu as pltpu
from jax.experimental.pallas import tpu_sc as plsc

BATCH = 16384
CH = 4096
NC = 2
NS = 16
NW = NC * NS
ROWS_PER_W = BATCH // NW
CHUNK = 4
NCHUNKS = ROWS_PER_W // CHUNK
LANES = 16


def _shuffle_sc(inputs, perm32):
    mesh = plsc.VectorSubcoreMesh(core_axis_name="c", subcore_axis_name="s")
    cp = pltpu.CompilerParams()
    if "needs_layout_passes" in pltpu.CompilerParams.__dataclass_fields__:
        cp = dataclasses.replace(cp, needs_layout_passes=False)

    @functools.partial(
        pl.kernel,
        mesh=mesh,
        compiler_params=cp,
        out_type=jax.ShapeDtypeStruct((BATCH, CH), jnp.float32),
        scratch_types=[
            pltpu.VMEM((CH,), jnp.int32),
            pltpu.VMEM((CHUNK, CH), jnp.float32),
            pltpu.VMEM((CHUNK, CH), jnp.float32),
            pltpu.VMEM((CHUNK, CH), jnp.float32),
            pltpu.VMEM((CHUNK, CH), jnp.float32),
            pltpu.SemaphoreType.DMA,
            pltpu.SemaphoreType.DMA,
            pltpu.SemaphoreType.DMA,
            pltpu.SemaphoreType.DMA,
        ],
    )
    def k(in_hbm, perm_hbm, out_hbm, perm_v, in_a, in_b, out_a, out_b,
          isem_a, isem_b, osem_a, osem_b):
        wid = lax.axis_index("s") * NC + lax.axis_index("c")
        base_row = wid * ROWS_PER_W
        pltpu.sync_copy(perm_hbm, perm_v)

        bufs = ((in_a, isem_a, out_a, osem_a), (in_b, isem_b, out_b, osem_b))

        def in_copy(c, buf, sem):
            return pltpu.make_async_copy(
                in_hbm.at[pl.ds(base_row + c * CHUNK, CHUNK)], buf, sem)

        def out_copy(c, buf, sem):
            return pltpu.make_async_copy(
                buf, out_hbm.at[pl.ds(base_row + c * CHUNK, CHUNK)], sem)

        in_copy(0, in_a, isem_a).start()
        in_copy(1, in_b, isem_b).start()

        @pl.loop(0, NCHUNKS // 2)
        def _(g):
            for b in range(2):
                in_v, isem, out_v, osem = bufs[b]
                c = g * 2 + b
                in_copy(c, in_v, isem).wait()

                @pl.when(g > 0)
                def _():
                    out_copy(c - 2, out_v, osem).wait()

                @plsc.parallel_loop(0, CH, LANES, unroll=4)
                def _(j):
                    pv = perm_v[pl.ds(j, LANES)]
                    for r in range(CHUNK):
                        rsplat = jnp.full((LANES,), r, jnp.int32)
                        g16 = plsc.load_gather(in_v, [rsplat, pv])
                        out_v[r, pl.ds(j, LANES)] = g16

                out_copy(c, out_v, osem).start()

                @pl.when(c + 2 < NCHUNKS)
                def _():
                    in_copy(c + 2, in_v, isem).start()

        out_copy(NCHUNKS - 2, out_a, osem_a).wait()
        out_copy(NCHUNKS - 1, out_b, osem_b).wait()

    return k(inputs, perm32)


def kernel(inputs, perm):
    perm32 = perm.astype(jnp.int32)
    shuffled = _shuffle_sc(inputs, perm32)
    logdet = jnp.zeros((BATCH,), dtype=inputs.dtype)
    return (shuffled, logdet)
